# baseline (speedup 1.0000x reference)
import functools

import jax
import jax.numpy as jnp
from jax import lax
from jax.experimental import pallas as pl
from jax.experimental.pallas import tpu as pltpu
from jax.experimental.pallas import tpu_sc as plsc

B = 16384
D = 64
NC, NS = 2, 16
NW = NC * NS
BPW = B // NW
BK = 512


def _gather_body(user_idx, item_idx, pos_idx, neg_idx,
                 user_tab, item_tab, tag_tab,
                 u_out, i_out, p_out, n_out,
                 idx_v, rows_v, sem):
    wid = lax.axis_index("s") * NC + lax.axis_index("c")
    base = wid * BPW
    jobs = ((user_idx, user_tab, u_out),
            (item_idx, item_tab, i_out),
            (pos_idx, tag_tab, p_out),
            (neg_idx, tag_tab, n_out))
    for idx_hbm, tab, out in jobs:
        pltpu.sync_copy(idx_hbm.at[pl.ds(base, BPW)], idx_v)
        pltpu.async_copy(tab.at[idx_v], rows_v, sem).wait()
        pltpu.sync_copy(rows_v, out.at[pl.ds(base, BPW)])


@jax.jit
def _gather(user, item, pos_tag, neg_tag, user_table, item_table, tag_table):
    mesh = plsc.VectorSubcoreMesh(core_axis_name="c", subcore_axis_name="s",
                                  num_cores=NC, num_subcores=NS)
    emb = jax.ShapeDtypeStruct((B, D), jnp.float32)
    run = pl.kernel(
        _gather_body,
        out_type=(emb, emb, emb, emb),
        mesh=mesh,
        scratch_types=[
            pltpu.VMEM((BPW,), jnp.int32),
            pltpu.VMEM((BPW, D), jnp.float32),
            pltpu.SemaphoreType.DMA,
        ],
        compiler_params=pltpu.CompilerParams(use_tc_tiling_on_sc=False),
    )
    return run(user, item, pos_tag, neg_tag, user_table, item_table, tag_table)


def _score_body(u_ref, i_ref, p_ref, n_ref, e_ref, f_ref, c_ref, out_ref):
    u_exp = jnp.dot(u_ref[...], e_ref[...], preferred_element_type=jnp.float32)
    i_exp = jnp.dot(i_ref[...], f_ref[...], preferred_element_type=jnp.float32)
    w = jnp.dot(u_exp * i_exp, c_ref[...], preferred_element_type=jnp.float32)
    d = p_ref[...] - n_ref[...]
    s = jnp.sum(w * d, axis=1)
    ls = jnp.minimum(s, 0.0) - jnp.log(1.0 + jnp.exp(-jnp.abs(s)))
    part = jnp.sum(ls)

    @pl.when(pl.program_id(0) == 0)
    def _init():
        out_ref[0, 0] = 0.0

    out_ref[0, 0] += part

    @pl.when(pl.program_id(0) == pl.num_programs(0) - 1)
    def _fin():
        out_ref[0, 0] = out_ref[0, 0] * (-1.0 / B)


@jax.jit
def _score(u_emb, i_emb, p_emb, n_emb, core_tensor):
    e_mat = jnp.repeat(jnp.eye(D, dtype=jnp.float32), D, axis=1)
    f_mat = jnp.tile(jnp.eye(D, dtype=jnp.float32), (1, D))
    c_mat = core_tensor.reshape(D * D, D)
    row = pl.BlockSpec((BK, D), lambda i: (i, 0))
    loss = pl.pallas_call(
        _score_body,
        grid=(B // BK,),
        in_specs=[
            row, row, row, row,
            pl.BlockSpec((D, D * D), lambda i: (0, 0)),
            pl.BlockSpec((D, D * D), lambda i: (0, 0)),
            pl.BlockSpec((D * D, D), lambda i: (0, 0)),
        ],
        out_specs=pl.BlockSpec((1, 1), lambda i: (0, 0),
                               memory_space=pltpu.SMEM),
        out_shape=jax.ShapeDtypeStruct((1, 1), jnp.float32),
    )(u_emb, i_emb, p_emb, n_emb, e_mat, f_mat, c_mat)
    return loss[0, 0]


def kernel(user, item, pos_tag, neg_tag, user_table, item_table,
           good_tag_table, core_tensor):
    u_emb, i_emb, p_emb, n_emb = _gather(
        user, item, pos_tag, neg_tag, user_table, item_table, good_tag_table)
    return _score(u_emb, i_emb, p_emb, n_emb, core_tensor)


# speedup vs baseline: 1.0938x; 1.0938x over previous
import functools

import jax
import jax.numpy as jnp
from jax import lax
from jax.experimental import pallas as pl
from jax.experimental.pallas import tpu as pltpu
from jax.experimental.pallas import tpu_sc as plsc

B = 16384
D = 64
NC, NS = 2, 16
NW = NC * NS
BPW = B // NW
BK = 1024


def _gather_body(user_idx, item_idx, pos_idx, neg_idx,
                 user_tab, item_tab, tag_tab,
                 u_out, i_out, p_out, n_out,
                 idx_v, rows_v, sem):
    wid = lax.axis_index("s") * NC + lax.axis_index("c")
    base = wid * BPW
    jobs = ((user_idx, user_tab, u_out),
            (item_idx, item_tab, i_out),
            (pos_idx, tag_tab, p_out),
            (neg_idx, tag_tab, n_out))
    for idx_hbm, tab, out in jobs:
        pltpu.sync_copy(idx_hbm.at[pl.ds(base, BPW)], idx_v)
        pltpu.async_copy(tab.at[idx_v], rows_v, sem).wait()
        pltpu.sync_copy(rows_v, out.at[pl.ds(base, BPW)])


@jax.jit
def _gather(user, item, pos_tag, neg_tag, user_table, item_table, tag_table):
    mesh = plsc.VectorSubcoreMesh(core_axis_name="c", subcore_axis_name="s",
                                  num_cores=NC, num_subcores=NS)
    emb = jax.ShapeDtypeStruct((B, D), jnp.float32)
    run = pl.kernel(
        _gather_body,
        out_type=(emb, emb, emb, emb),
        mesh=mesh,
        scratch_types=[
            pltpu.VMEM((BPW,), jnp.int32),
            pltpu.VMEM((BPW, D), jnp.float32),
            pltpu.SemaphoreType.DMA,
        ],
        compiler_params=pltpu.CompilerParams(use_tc_tiling_on_sc=False),
    )
    return run(user, item, pos_tag, neg_tag, user_table, item_table, tag_table)


def _score_body(u_ref, i_ref, p_ref, n_ref, e_ref, c_ref, out_ref):
    u_bf = u_ref[...].astype(jnp.bfloat16)
    i_bf = i_ref[...].astype(jnp.bfloat16)
    u_exp = jnp.dot(u_bf, e_ref[...],
                    preferred_element_type=jnp.float32).astype(jnp.bfloat16)
    i_tiled = pltpu.repeat(i_bf, D, axis=1)
    p_outer = u_exp * i_tiled
    w = jnp.dot(p_outer, c_ref[...], preferred_element_type=jnp.float32)
    d = p_ref[...] - n_ref[...]
    s = jnp.sum(w * d, axis=1)
    ls = jnp.minimum(s, 0.0) - jnp.log(1.0 + jnp.exp(-jnp.abs(s)))
    part = jnp.sum(ls)

    @pl.when(pl.program_id(0) == 0)
    def _init():
        out_ref[0, 0] = 0.0

    out_ref[0, 0] += part

    @pl.when(pl.program_id(0) == pl.num_programs(0) - 1)
    def _fin():
        out_ref[0, 0] = out_ref[0, 0] * (-1.0 / B)


@jax.jit
def _score(u_emb, i_emb, p_emb, n_emb, core_tensor):
    e_mat = jnp.repeat(jnp.eye(D, dtype=jnp.bfloat16), D, axis=1)
    c_mat = core_tensor.reshape(D * D, D).astype(jnp.bfloat16)
    row = pl.BlockSpec((BK, D), lambda i: (i, 0))
    loss = pl.pallas_call(
        _score_body,
        grid=(B // BK,),
        in_specs=[
            row, row, row, row,
            pl.BlockSpec((D, D * D), lambda i: (0, 0)),
            pl.BlockSpec((D * D, D), lambda i: (0, 0)),
        ],
        out_specs=pl.BlockSpec((1, 1), lambda i: (0, 0),
                               memory_space=pltpu.SMEM),
        out_shape=jax.ShapeDtypeStruct((1, 1), jnp.float32),
    )(u_emb, i_emb, p_emb, n_emb, e_mat, c_mat)
    return loss[0, 0]


def kernel(user, item, pos_tag, neg_tag, user_table, item_table,
           good_tag_table, core_tensor):
    u_emb, i_emb, p_emb, n_emb = _gather(
        user, item, pos_tag, neg_tag, user_table, item_table, good_tag_table)
    return _score(u_emb, i_emb, p_emb, n_emb, core_tensor)
